# baseline (speedup 1.0000x reference)
import functools

import jax
import jax.numpy as jnp
from jax import lax
from jax.experimental import pallas as pl
from jax.experimental.pallas import tpu as pltpu
from jax.experimental.pallas import tpu_sc as plsc

D_MODEL = 64
NUM_CORES = 2
NUM_SUBCORES = 16
NUM_WORKERS = NUM_CORES * NUM_SUBCORES
BB = 512


@functools.cache
def _build_lookup(n_t: int, n_b: int):
    assert n_b % BB == 0
    units_per_t = n_b // BB
    n_units = n_t * units_per_t
    assert n_units % NUM_WORKERS == 0
    units_per_worker = n_units // NUM_WORKERS
    mesh = plsc.VectorSubcoreMesh(
        core_axis_name="c",
        subcore_axis_name="s",
        num_cores=NUM_CORES,
        num_subcores=NUM_SUBCORES,
    )

    @functools.partial(
        pl.kernel,
        out_type=jax.ShapeDtypeStruct((n_t, D_MODEL, n_b), jnp.float32),
        mesh=mesh,
        scratch_types=[
            pltpu.VMEM((BB,), jnp.int32),
            pltpu.VMEM((BB,), jnp.int32),
            pltpu.VMEM((BB, 2 * D_MODEL), jnp.float32),
            pltpu.VMEM((D_MODEL, BB), jnp.float32),
            pltpu.SemaphoreType.DMA,
        ],
        compiler_params=pltpu.CompilerParams(
            use_tc_tiling_on_sc=True, needs_layout_passes=False
        ),
    )
    def lookup_kernel(table2, ids_t, out_t, kv, h64v, rows2, outT, sem):
        wid = lax.axis_index("s") * NUM_CORES + lax.axis_index("c")

        def unit(u, carry):
            t = u // units_per_t
            b0 = (u % units_per_t) * BB
            pltpu.sync_copy(ids_t.at[t, pl.ds(b0, BB)], kv)

            def split(j, c):
                v = kv[pl.ds(j * 16, 16)]
                h64v[pl.ds(j * 16, 16)] = (v & 1) * D_MODEL
                kv[pl.ds(j * 16, 16)] = v >> 1
                return c

            lax.fori_loop(0, BB // 16, split, 0, unroll=True)
            pltpu.async_copy(table2.at[kv], rows2, sem).wait()

            def cbody(c, carry2):
                def jbody(j, c2):
                    ids0 = lax.iota(jnp.int32, 16) + j * 16
                    ids1 = h64v[pl.ds(j * 16, 16)] + c
                    outT[c, pl.ds(j * 16, 16)] = plsc.load_gather(
                        rows2, [ids0, ids1]
                    )
                    return c2

                return lax.fori_loop(0, BB // 16, jbody, carry2, unroll=True)

            lax.fori_loop(0, D_MODEL, cbody, 0)
            pltpu.sync_copy(outT, out_t.at[t, :, pl.ds(b0, BB)])
            return carry

        lax.fori_loop(
            wid * units_per_worker,
            (wid + 1) * units_per_worker,
            unit,
            0,
        )

    return lookup_kernel


def kernel(time_ids, pe_weight):
    n_b, n_t = time_ids.shape
    ids_t = time_ids.astype(jnp.int32).T
    table2 = pe_weight.reshape(pe_weight.shape[0] // 2, 2 * D_MODEL)
    out_t = _build_lookup(n_t, n_b)(table2, ids_t)
    return out_t.transpose(2, 0, 1)


# speedup vs baseline: 1.3206x; 1.3206x over previous
import functools

import jax
import jax.numpy as jnp
from jax import lax
from jax.experimental import pallas as pl
from jax.experimental.pallas import tpu as pltpu
from jax.experimental.pallas import tpu_sc as plsc

D_MODEL = 64
NUM_CORES = 2
NUM_SUBCORES = 16
NUM_WORKERS = NUM_CORES * NUM_SUBCORES
BB = 512


@functools.cache
def _build_lookup(n_t: int, n_b: int):
    assert n_b % BB == 0
    units_per_t = n_b // BB
    n_units = n_t * units_per_t
    assert n_units % NUM_WORKERS == 0
    units_per_worker = n_units // NUM_WORKERS
    mesh = plsc.VectorSubcoreMesh(
        core_axis_name="c",
        subcore_axis_name="s",
        num_cores=NUM_CORES,
        num_subcores=NUM_SUBCORES,
    )

    @functools.partial(
        pl.kernel,
        out_type=jax.ShapeDtypeStruct((n_t, D_MODEL, n_b), jnp.float32),
        mesh=mesh,
        scratch_types=[
            pltpu.VMEM((BB,), jnp.int32),
            pltpu.VMEM((BB,), jnp.int32),
            pltpu.VMEM((BB, 2 * D_MODEL), jnp.float32),
            pltpu.VMEM((D_MODEL, BB), jnp.float32),
            pltpu.SemaphoreType.DMA,
        ],
        compiler_params=pltpu.CompilerParams(
            use_tc_tiling_on_sc=True, needs_layout_passes=False
        ),
    )
    def lookup_kernel(table2, ids_t, out_t, kv, h64v, rows2, outT, sem):
        wid = lax.axis_index("s") * NUM_CORES + lax.axis_index("c")

        def unit(u, carry):
            t = u // units_per_t
            b0 = (u % units_per_t) * BB
            pltpu.sync_copy(ids_t.at[t, pl.ds(b0, BB)], kv)

            def split(j, c):
                v = kv[pl.ds(j * 16, 16)]
                h64v[pl.ds(j * 16, 16)] = (v & 1) * D_MODEL
                kv[pl.ds(j * 16, 16)] = v >> 1
                return c

            lax.fori_loop(0, BB // 16, split, 0, unroll=True)
            pltpu.async_copy(table2.at[kv], rows2, sem).wait()

            def jbody(j, carry2):
                ids0 = lax.iota(jnp.int32, 16) + j * 16
                h64c = h64v[pl.ds(j * 16, 16)]
                for c in range(D_MODEL):
                    outT[c, pl.ds(j * 16, 16)] = plsc.load_gather(
                        rows2, [ids0, h64c + c]
                    )
                return carry2

            lax.fori_loop(0, BB // 16, jbody, 0)
            pltpu.sync_copy(outT, out_t.at[t, :, pl.ds(b0, BB)])
            return carry

        lax.fori_loop(
            wid * units_per_worker,
            (wid + 1) * units_per_worker,
            unit,
            0,
        )

    return lookup_kernel


def kernel(time_ids, pe_weight):
    n_b, n_t = time_ids.shape
    ids_t = time_ids.astype(jnp.int32).T
    table2 = pe_weight.reshape(pe_weight.shape[0] // 2, 2 * D_MODEL)
    out_t = _build_lookup(n_t, n_b)(table2, ids_t)
    return out_t.transpose(2, 0, 1)


# speedup vs baseline: 1.4425x; 1.0923x over previous
import functools

import jax
import jax.numpy as jnp
from jax import lax
from jax.experimental import pallas as pl
from jax.experimental.pallas import tpu as pltpu
from jax.experimental.pallas import tpu_sc as plsc

D_MODEL = 64
NUM_CORES = 2
NUM_SUBCORES = 16
NUM_WORKERS = NUM_CORES * NUM_SUBCORES
BB = 512


@functools.cache
def _build_lookup(n_t: int, n_b: int):
    assert n_b % BB == 0
    units_per_t = n_b // BB
    n_units = n_t * units_per_t
    assert n_units % NUM_WORKERS == 0
    units_per_worker = n_units // NUM_WORKERS
    mesh = plsc.VectorSubcoreMesh(
        core_axis_name="c",
        subcore_axis_name="s",
        num_cores=NUM_CORES,
        num_subcores=NUM_SUBCORES,
    )

    @functools.partial(
        pl.kernel,
        out_type=jax.ShapeDtypeStruct((n_t, D_MODEL, n_b), jnp.float32),
        mesh=mesh,
        scratch_types=[
            pltpu.VMEM((BB,), jnp.int32),
            pltpu.VMEM((BB,), jnp.int32),
            pltpu.VMEM((BB, 2 * D_MODEL), jnp.float32),
            pltpu.VMEM((D_MODEL, BB), jnp.float32),
            pltpu.SemaphoreType.DMA,
        ],
        compiler_params=pltpu.CompilerParams(
            use_tc_tiling_on_sc=True, needs_layout_passes=False
        ),
    )
    def lookup_kernel(table2, ids_t, out_t, kv, h64v, rows2, outT, sem):
        wid = lax.axis_index("s") * NUM_CORES + lax.axis_index("c")

        def unit(u, carry):
            t = u // units_per_t
            b0 = (u % units_per_t) * BB
            pltpu.sync_copy(ids_t.at[t, pl.ds(b0, BB)], kv)

            def split(j, c):
                v = kv[pl.ds(j * 16, 16)]
                h64v[pl.ds(j * 16, 16)] = (v & 1) * D_MODEL
                kv[pl.ds(j * 16, 16)] = v >> 1
                return c

            lax.fori_loop(0, BB // 16, split, 0, unroll=True)
            pltpu.async_copy(table2.at[kv], rows2, sem).wait()

            @plsc.parallel_loop(0, BB // 16, step=1, unroll=2)
            def jbody(j):
                ids0 = lax.iota(jnp.int32, 16) + j * 16
                h64c = h64v[pl.ds(j * 16, 16)]
                for c in range(D_MODEL):
                    outT[c, pl.ds(j * 16, 16)] = plsc.load_gather(
                        rows2, [ids0, h64c + c]
                    )
            pltpu.sync_copy(outT, out_t.at[t, :, pl.ds(b0, BB)])
            return carry

        lax.fori_loop(
            wid * units_per_worker,
            (wid + 1) * units_per_worker,
            unit,
            0,
        )

    return lookup_kernel


def kernel(time_ids, pe_weight):
    n_b, n_t = time_ids.shape
    ids_t = time_ids.astype(jnp.int32).T
    table2 = pe_weight.reshape(pe_weight.shape[0] // 2, 2 * D_MODEL)
    out_t = _build_lookup(n_t, n_b)(table2, ids_t)
    return out_t.transpose(2, 0, 1)


# speedup vs baseline: 2.2482x; 1.5586x over previous
import functools

import jax
import jax.numpy as jnp
from jax import lax
from jax.experimental import pallas as pl
from jax.experimental.pallas import tpu as pltpu
from jax.experimental.pallas import tpu_sc as plsc

D_MODEL = 64
NUM_CORES = 2
NUM_SUBCORES = 16
NUM_WORKERS = NUM_CORES * NUM_SUBCORES
CHUNK = 512
NBUF = 3


@functools.cache
def _build_gather(n_rows: int):
    assert n_rows % (NUM_WORKERS * CHUNK) == 0
    rows_per_worker = n_rows // NUM_WORKERS
    n_steps = rows_per_worker // CHUNK
    mesh = plsc.VectorSubcoreMesh(
        core_axis_name="c",
        subcore_axis_name="s",
        num_cores=NUM_CORES,
        num_subcores=NUM_SUBCORES,
    )

    @functools.partial(
        pl.kernel,
        out_type=jax.ShapeDtypeStruct((n_rows, D_MODEL), jnp.float32),
        mesh=mesh,
        scratch_types=[
            pltpu.VMEM((n_steps, CHUNK), jnp.int32),
            [pltpu.VMEM((CHUNK, D_MODEL), jnp.float32) for _ in range(NBUF)],
            [pltpu.SemaphoreType.DMA for _ in range(NBUF)],
            [pltpu.SemaphoreType.DMA for _ in range(NBUF)],
        ],
        compiler_params=pltpu.CompilerParams(use_tc_tiling_on_sc=False),
    )
    def gather_kernel(table_hbm, idx_hbm, out_hbm, ids_v, row_bufs, gsems, wsems):
        wid = lax.axis_index("s") * NUM_CORES + lax.axis_index("c")
        base = wid * rows_per_worker

        pltpu.sync_copy(idx_hbm.at[pl.ds(wid * n_steps, n_steps)], ids_v)

        def fire(gi, b):
            pltpu.async_copy(table_hbm.at[ids_v.at[gi]], row_bufs[b], gsems[b])

        def drain(di, b):
            pltpu.make_async_copy(
                table_hbm.at[ids_v.at[di]], row_bufs[b], gsems[b]
            ).wait()
            dst = out_hbm.at[pl.ds(base + di * CHUNK, CHUNK)]
            pltpu.async_copy(row_bufs[b], dst, wsems[b])

        for gi in range(n_steps + NBUF - 1):
            b = gi % NBUF
            if gi < n_steps:
                if gi >= NBUF:
                    dst = out_hbm.at[pl.ds(base + (gi - NBUF) * CHUNK, CHUNK)]
                    pltpu.make_async_copy(row_bufs[b], dst, wsems[b]).wait()
                fire(gi, b)
            di = gi - (NBUF - 1)
            if 0 <= di < n_steps:
                drain(di, di % NBUF)

        for di in range(n_steps - NBUF, n_steps):
            b = di % NBUF
            dst = out_hbm.at[pl.ds(base + di * CHUNK, CHUNK)]
            pltpu.make_async_copy(row_bufs[b], dst, wsems[b]).wait()

    return gather_kernel


def kernel(time_ids, pe_weight):
    shape = time_ids.shape
    idx = time_ids.reshape(-1).astype(jnp.int32)
    n_rows = idx.shape[0]
    idx2 = idx.reshape(n_rows // CHUNK, CHUNK)
    out = _build_gather(n_rows)(pe_weight, idx2)
    return out.reshape(*shape, D_MODEL)


# speedup vs baseline: 2.2524x; 1.0019x over previous
import functools

import jax
import jax.numpy as jnp
from jax import lax
from jax.experimental import pallas as pl
from jax.experimental.pallas import tpu as pltpu
from jax.experimental.pallas import tpu_sc as plsc

D_MODEL = 64
NUM_CORES = 2
NUM_SUBCORES = 16
NUM_WORKERS = NUM_CORES * NUM_SUBCORES
CHUNK = 512
NBUF = 3


@functools.cache
def _build_gather(n_rows: int):
    assert n_rows % (NUM_WORKERS * CHUNK) == 0
    rows_per_worker = n_rows // NUM_WORKERS
    n_steps = rows_per_worker // CHUNK
    mesh = plsc.VectorSubcoreMesh(
        core_axis_name="c",
        subcore_axis_name="s",
        num_cores=NUM_CORES,
        num_subcores=NUM_SUBCORES,
    )

    @functools.partial(
        pl.kernel,
        out_type=jax.ShapeDtypeStruct((n_rows, D_MODEL), jnp.float32),
        mesh=mesh,
        scratch_types=[
            pltpu.VMEM((rows_per_worker,), jnp.int32),
            [pltpu.VMEM((CHUNK, D_MODEL), jnp.float32) for _ in range(NBUF)],
            [pltpu.SemaphoreType.DMA for _ in range(NBUF)],
            [pltpu.SemaphoreType.DMA for _ in range(NBUF)],
        ],
        compiler_params=pltpu.CompilerParams(use_tc_tiling_on_sc=False),
    )
    def gather_kernel(table_hbm, idx_hbm, out_hbm, ids_v, row_bufs, gsems, wsems):
        wid = lax.axis_index("s") * NUM_CORES + lax.axis_index("c")
        base = wid * rows_per_worker

        pltpu.sync_copy(idx_hbm.at[pl.ds(base, rows_per_worker)], ids_v)

        def fire(gi, b):
            idx_v = ids_v.at[pl.ds(gi * CHUNK, CHUNK)]
            pltpu.async_copy(table_hbm.at[idx_v], row_bufs[b], gsems[b])

        def drain(di, b):
            idx_v = ids_v.at[pl.ds(di * CHUNK, CHUNK)]
            pltpu.make_async_copy(
                table_hbm.at[idx_v], row_bufs[b], gsems[b]
            ).wait()
            dst = out_hbm.at[pl.ds(base + di * CHUNK, CHUNK)]
            pltpu.async_copy(row_bufs[b], dst, wsems[b])

        for gi in range(n_steps + NBUF - 1):
            b = gi % NBUF
            if gi < n_steps:
                if gi >= NBUF:
                    dst = out_hbm.at[pl.ds(base + (gi - NBUF) * CHUNK, CHUNK)]
                    pltpu.make_async_copy(row_bufs[b], dst, wsems[b]).wait()
                fire(gi, b)
            di = gi - (NBUF - 1)
            if 0 <= di < n_steps:
                drain(di, di % NBUF)

        for di in range(n_steps - NBUF, n_steps):
            b = di % NBUF
            dst = out_hbm.at[pl.ds(base + di * CHUNK, CHUNK)]
            pltpu.make_async_copy(row_bufs[b], dst, wsems[b]).wait()

    return gather_kernel


def kernel(time_ids, pe_weight):
    shape = time_ids.shape
    idx = time_ids.reshape(-1).astype(jnp.int32)
    n_rows = idx.shape[0]
    out = _build_gather(n_rows)(pe_weight, idx)
    return out.reshape(*shape, D_MODEL)
